# Initial kernel scaffold; baseline (speedup 1.0000x reference)
import jax
import jax.numpy as jnp
from jax.experimental import pallas as pl


def kernel(center_vectors, outside_vectors, center_word_index, outside_word_indices, negative_samples):
    raise NotImplementedError("write your pallas kernel here")


# speedup vs baseline: 1.2158x; 1.2158x over previous
import functools

import jax
import jax.numpy as jnp
from jax import lax
from jax.experimental import pallas as pl
from jax.experimental.pallas import tpu as pltpu
from jax.experimental.pallas import tpu_sc as plsc

_NC = 2
_NS = 16
_NW = _NC * _NS

_D = 32
_O = 20
_K = 10
_SLOTS = 224
_G = _SLOTS // 16
_HALF = _SLOTS // 2


def _sc_dots(center_vectors, outside_vectors, idx_flat, center_idx):
    B = center_idx.shape[0]
    epw = B // _NW

    mesh = plsc.VectorSubcoreMesh(core_axis_name="c", subcore_axis_name="s")

    @functools.partial(
        pl.kernel,
        out_type=jax.ShapeDtypeStruct((B * _SLOTS,), jnp.float32),
        mesh=mesh,
        compiler_params=pltpu.CompilerParams(
            needs_layout_passes=False, use_tc_tiling_on_sc=False),
        scratch_types=[
            pltpu.VMEM((epw * _SLOTS,), jnp.int32),
            pltpu.VMEM((epw,), jnp.int32),
            pltpu.VMEM((epw, _D), jnp.float32),
            pltpu.VMEM((2, _SLOTS, _D), jnp.float32),
            pltpu.VMEM((epw * _SLOTS,), jnp.float32),
            pltpu.SemaphoreType.DMA,
            pltpu.SemaphoreType.DMA,
            pltpu.SemaphoreType.DMA,
        ],
    )
    def k(cvec_hbm, ovec_hbm, idx_hbm, cidx_hbm, out_hbm,
          idx_v, cidx_v, z_v, rows_v, out_v, semz, sem0, sem1):
        wid = lax.axis_index("s") * _NC + lax.axis_index("c")
        ebase = wid * epw

        pltpu.sync_copy(idx_hbm.at[pl.ds(ebase * _SLOTS, epw * _SLOTS)], idx_v)
        pltpu.sync_copy(cidx_hbm.at[pl.ds(ebase, epw)], cidx_v)
        pltpu.async_copy(cvec_hbm.at[cidx_v], z_v, semz).wait()

        def start(e, buf, sem):
            for h in range(2):
                pltpu.async_copy(
                    ovec_hbm.at[idx_v.at[pl.ds(e * _SLOTS + h * _HALF, _HALF)]],
                    rows_v.at[buf, pl.ds(h * _HALF, _HALF)],
                    sem,
                )

        def drain(buf, sem):
            for h in range(2):
                pltpu.make_async_copy(
                    ovec_hbm.at[pl.ds(0, _HALF)],
                    rows_v.at[buf, pl.ds(h * _HALF, _HALF)],
                    sem,
                ).wait()

        iota16 = lax.iota(jnp.int32, 16)
        row_ids = [iota16 + g * 16 for g in range(_G)]

        def compute(e, buf):
            rows = rows_v.at[buf]
            zhalves = [z_v[e, pl.ds(0, 16)], z_v[e, pl.ds(16, 16)]]
            accs = [jnp.zeros((16,), jnp.float32)] * _G
            for d in range(_D):
                zd = zhalves[d // 16][d % 16]
                dvec = jnp.full((16,), d, jnp.int32)
                accs = [
                    accs[g] + plsc.load_gather(rows, [row_ids[g], dvec]) * zd
                    for g in range(_G)
                ]
            for g in range(_G):
                out_v[pl.ds(e * _SLOTS + g * 16, 16)] = accs[g]

        start(0, 0, sem0)
        start(1, 1, sem1)

        def body(i, carry):
            for sub, sem in ((0, sem0), (1, sem1)):
                e = 2 * i + sub
                drain(sub, sem)
                compute(e, sub)

                @pl.when(i < epw // 2 - 1)
                def _():
                    start(e + 2, sub, sem)
            return carry

        lax.fori_loop(0, epw // 2, body, 0)

        pltpu.sync_copy(out_v, out_hbm.at[pl.ds(ebase * _SLOTS, epw * _SLOTS)])

    return k(center_vectors, outside_vectors, idx_flat, center_idx)


def _tc_loss(dots, idx_all, own_all):
    B = dots.shape[0]
    tb = 512

    def body(dots_ref, idx_ref, own_ref, out_ref):
        d = dots_ref[...]
        valid = (idx_ref[...] != 0) & (own_ref[...] != 0)
        col = lax.broadcasted_iota(jnp.int32, d.shape, 1)
        t = jnp.where(col < _O, d, -d)
        ls = jnp.minimum(t, 0.0) - jnp.log1p(jnp.exp(-jnp.abs(t)))
        out_ref[...] = -jnp.sum(jnp.where(valid, ls, 0.0), axis=1)

    return pl.pallas_call(
        body,
        out_shape=jax.ShapeDtypeStruct((B,), jnp.float32),
        grid=(B // tb,),
        in_specs=[
            pl.BlockSpec((tb, _SLOTS), lambda i: (i, 0)),
            pl.BlockSpec((tb, _SLOTS), lambda i: (i, 0)),
            pl.BlockSpec((tb, _SLOTS), lambda i: (i, 0)),
        ],
        out_specs=pl.BlockSpec((tb,), lambda i: (i,)),
    )(dots, idx_all, own_all)


def kernel(center_vectors, outside_vectors, center_word_index,
           outside_word_indices, negative_samples):
    B, O = outside_word_indices.shape
    K = negative_samples.shape[-1]

    neg2 = negative_samples.reshape(B, O * K)
    pad = jnp.zeros((B, _SLOTS - O - O * K), jnp.int32)
    idx_all = jnp.concatenate([outside_word_indices, neg2, pad], axis=1)
    own_all = jnp.concatenate(
        [outside_word_indices,
         jnp.repeat(outside_word_indices, K, axis=1),
         pad],
        axis=1,
    )

    dots = _sc_dots(center_vectors, outside_vectors,
                    idx_all.reshape(-1), center_word_index)
    return _tc_loss(dots.reshape(B, _SLOTS), idx_all, own_all)
